# baseline (speedup 1.0000x reference)
import functools

import jax
import jax.numpy as jnp
from jax import lax
from jax.experimental import pallas as pl
from jax.experimental.pallas import tpu as pltpu
from jax.experimental.pallas import tpu_sc as plsc


def _sc_gather(table, hist_idx, src_off, mov_off, mov_idx, *, seq, seq_pad,
               hist_chunk, mov_chunk):
    n_hist = hist_idx.shape[0]
    n_mov = mov_idx.shape[0]
    d = table.shape[1]
    info = plsc.get_sparse_core_info()
    nw = info.num_cores * info.num_subcores
    nl = info.num_lanes
    hist_per_w = n_hist // nw
    mov_per_w = n_mov // nw
    n_hchunks = hist_per_w // hist_chunk
    n_mchunks = mov_per_w // mov_chunk
    grp = 4 * seq
    g_per_chunk = hist_chunk // grp
    rows_pq = hist_chunk // 4
    n_hrows = (n_hist // grp) * seq_pad
    mq = mov_chunk // 4

    mesh = plsc.VectorSubcoreMesh(core_axis_name="c", subcore_axis_name="s")

    @functools.partial(
        pl.kernel,
        mesh=mesh,
        out_type=(
            jax.ShapeDtypeStruct((n_hrows, 4 * d), jnp.float32),
            jax.ShapeDtypeStruct((n_mov // 4, 4 * d), jnp.float32),
        ),
        scratch_types=[
            pltpu.VMEM((hist_per_w,), jnp.int32),
            pltpu.VMEM((hist_per_w,), jnp.int32),
            pltpu.VMEM((hist_chunk,), jnp.int32),
            pltpu.VMEM((4, rows_pq, d), jnp.float32),
            pltpu.VMEM((mov_chunk,), jnp.int32),
            pltpu.VMEM((mov_chunk,), jnp.int32),
            pltpu.VMEM((mov_chunk,), jnp.int32),
            pltpu.VMEM((4, mq, d), jnp.float32),
            pltpu.SemaphoreType.DMA,
        ],
        compiler_params=pltpu.CompilerParams(
            use_tc_tiling_on_sc=False, needs_layout_passes=False),
    )
    def k(table_hbm, hidx_hbm, soff_hbm, msoff_hbm, midx_hbm,
          hist_out, mov_out,
          nat_v, soff_v, pidx_v, hrows_v, mnat_v, msoff_v, mpidx_v,
          mrows_v, sem):
        wid = lax.axis_index("s") * info.num_cores + lax.axis_index("c")
        hbase = wid * hist_per_w
        gbase = wid * (hist_per_w // grp)
        mbase = wid * mov_chunk

        pltpu.sync_copy(hidx_hbm.at[pl.ds(hbase, hist_per_w)], nat_v)
        pltpu.sync_copy(soff_hbm, soff_v)
        pltpu.sync_copy(msoff_hbm, msoff_v)

        def hist_body(i, _):
            local = i * hist_chunk

            def perm_body(v, _):
                p = pl.ds(local + v * nl, nl)
                pidx_v[pl.ds(v * nl, nl)] = plsc.load_gather(
                    nat_v, [soff_v[p]])
                return 0

            lax.fori_loop(0, hist_chunk // nl, perm_body, 0)
            cps = [pltpu.async_copy(
                table_hbm.at[pidx_v.at[pl.ds(r * rows_pq, rows_pq)]],
                hrows_v.at[r], sem) for r in range(4)]
            for cp in cps:
                cp.wait()
            for kk in range(g_per_chunk):
                base = (gbase + i * g_per_chunk + kk) * seq_pad
                for r in range(4):
                    pltpu.sync_copy(
                        hrows_v.at[r, pl.ds(kk * seq, seq)],
                        hist_out.at[pl.ds(base, seq), pl.ds(r * d, d)])
            return 0

        lax.fori_loop(0, n_hchunks, hist_body, 0)

        def mov_body(i, _):
            pltpu.sync_copy(midx_hbm.at[pl.ds(mbase, mov_chunk)], mnat_v)

            def mperm_body(v, _):
                mpidx_v[pl.ds(v * nl, nl)] = plsc.load_gather(
                    mnat_v, [msoff_v[pl.ds(v * nl, nl)]])
                return 0

            lax.fori_loop(0, mov_chunk // nl, mperm_body, 0)
            cps = [pltpu.async_copy(
                table_hbm.at[mpidx_v.at[pl.ds(r * mq, mq)]],
                mrows_v.at[r], sem) for r in range(4)]
            for cp in cps:
                cp.wait()
            for r in range(4):
                pltpu.sync_copy(
                    mrows_v.at[r],
                    mov_out.at[pl.ds(wid * mq, mq), pl.ds(r * d, d)])
            return 0

        lax.fori_loop(0, n_mchunks, mov_body, 0)

    return k(table, hist_idx, src_off, mov_off, mov_idx)


def _tc_body(hist_ref, rat_ref, mov_ref, w1_ref, b1_ref, w2_ref, b2_ref,
             w3_ref, b3_ref, out_ref, *, gsize, seq, seq_pad, dim):
    f32 = jnp.float32
    x = hist_ref[...]
    r = jnp.transpose(rat_ref[...], (0, 2, 1))

    lanes = 4 * dim
    lane_id = lax.broadcasted_iota(jnp.int32, (4, lanes), 1)
    row_id = lax.broadcasted_iota(jnp.int32, (4, lanes), 0)
    sel = jnp.where(lane_id // dim == row_id, f32(1), f32(0))
    r_exp = jnp.dot(r.reshape(gsize * seq, 4), sel,
                    preferred_element_type=f32).reshape(gsize, seq, lanes)
    if seq_pad > seq:
        r_exp = jnp.concatenate(
            [r_exp, jnp.zeros((gsize, seq_pad - seq, lanes), f32)], axis=1)

    inf = f32(jnp.inf)
    l_id = lax.broadcasted_iota(jnp.int32, (gsize, seq_pad, lanes), 1)
    valid = l_id < seq
    wf = x * r_exp
    w = jnp.where(valid, wf, inf)

    mn = jnp.min(w, axis=1)
    mx = jnp.max(jnp.where(valid, wf, -inf), axis=1)
    mean = jnp.sum(jnp.where(valid, wf, f32(0)), axis=1) * f32(1.0 / seq)

    c = jnp.zeros((gsize, seq_pad, lanes), dtype=f32)
    for j in range(seq):
        c = c + jnp.where(w[:, j:j + 1, :] <= w, f32(1), f32(0))
    lo_rank = f32((seq - 1) // 2 + 1)
    hi_rank = f32(seq // 2 + 1)
    s_lo = jnp.min(jnp.where(c >= lo_rank, w, inf), axis=1)
    s_hi = jnp.min(jnp.where(c >= hi_rank, w, inf), axis=1)
    med = f32(0.5) * (s_lo + s_hi)

    def unfold(s):
        parts = [s[:, rr * dim:(rr + 1) * dim].reshape(gsize, 1, dim)
                 for rr in range(4)]
        return jnp.concatenate(parts, axis=1).reshape(4 * gsize, dim)

    u = jnp.concatenate(
        [unfold(mn), unfold(mx), unfold(mean), unfold(med)], axis=1)
    u = u * lax.rsqrt(jnp.sum(u * u, axis=1, keepdims=True))

    xin = jnp.concatenate([u, unfold(mov_ref[...])], axis=1)
    h = jnp.dot(xin, w1_ref[...], preferred_element_type=f32) + b1_ref[...]
    h = jnp.maximum(h, f32(0))
    h = jnp.dot(h, w2_ref[...], preferred_element_type=f32) + b2_ref[...]
    h = jnp.maximum(h, f32(0))
    o = jnp.dot(h, w3_ref[...], preferred_element_type=f32) + b3_ref[...]
    out_ref[...] = f32(1) / (f32(1) + jnp.exp(-o))


def _tc_pool_mlp(hist_fold, rat_fold, mov_fold, W1, b1, W2, b2, W3, b3,
                 *, gsize):
    gtot, seq_pad, lanes = hist_fold.shape
    seq = rat_fold.shape[2]
    dim = lanes // 4
    batch = 4 * gtot
    nblocks = gtot // gsize
    body = functools.partial(_tc_body, gsize=gsize, seq=seq,
                             seq_pad=seq_pad, dim=dim)
    return pl.pallas_call(
        body,
        grid=(nblocks,),
        in_specs=[
            pl.BlockSpec((gsize, seq_pad, lanes), lambda i: (i, 0, 0)),
            pl.BlockSpec((gsize, 4, seq), lambda i: (i, 0, 0)),
            pl.BlockSpec((gsize, lanes), lambda i: (i, 0)),
            pl.BlockSpec(W1.shape, lambda i: (0, 0)),
            pl.BlockSpec((1, W1.shape[1]), lambda i: (0, 0)),
            pl.BlockSpec(W2.shape, lambda i: (0, 0)),
            pl.BlockSpec((1, W2.shape[1]), lambda i: (0, 0)),
            pl.BlockSpec(W3.shape, lambda i: (0, 0)),
            pl.BlockSpec((1, 1), lambda i: (0, 0)),
        ],
        out_specs=pl.BlockSpec((4 * gsize, 1), lambda i: (i, 0)),
        out_shape=jax.ShapeDtypeStruct((batch, 1), jnp.float32),
        compiler_params=pltpu.CompilerParams(
            dimension_semantics=("parallel",)),
    )(hist_fold, rat_fold, mov_fold, W1, b1.reshape(1, -1), W2,
      b2.reshape(1, -1), W3, b3.reshape(1, 1))


def kernel(user_hist_indices, user_hist_ratings, movie_indices, movie_table,
           W1, b1, W2, b2, W3, b3):
    batch, seq = user_hist_indices.shape
    dim = movie_table.shape[1]
    g = batch // 4

    rat_fold = user_hist_ratings.reshape(g, 4, seq)

    info = plsc.get_sparse_core_info()
    nw = info.num_cores * info.num_subcores
    per_w = (batch * seq) // nw
    hist_chunk = 1600
    gpc = hist_chunk // (4 * seq)
    n = jnp.arange(per_w, dtype=jnp.int32)
    ci = n // hist_chunk
    rr = (n % hist_chunk) // (gpc * seq)
    gg = (n % (gpc * seq)) // seq
    ll = n % seq
    src_off = ((ci * gpc + gg) * 4 + rr) * seq + ll

    mov_chunk = batch // nw
    m = jnp.arange(mov_chunk, dtype=jnp.int32)
    mov_off = (m % (mov_chunk // 4)) * 4 + m // (mov_chunk // 4)

    seq_pad = (seq + 7) // 8 * 8
    hist_rows, mov_emb = _sc_gather(
        movie_table, user_hist_indices.astype(jnp.int32).reshape(-1),
        src_off, mov_off, movie_indices.astype(jnp.int32),
        seq=seq, seq_pad=seq_pad, hist_chunk=hist_chunk, mov_chunk=mov_chunk)
    hist_fold = hist_rows.reshape(g, seq_pad, 4 * dim)
    mov_fold = mov_emb

    return _tc_pool_mlp(hist_fold, rat_fold, mov_fold,
                        W1, b1, W2, b2, W3, b3, gsize=64)


# speedup vs baseline: 1.0529x; 1.0529x over previous
import functools

import jax
import jax.numpy as jnp
from jax import lax
from jax.experimental import pallas as pl
from jax.experimental.pallas import tpu as pltpu
from jax.experimental.pallas import tpu_sc as plsc


def _sc_gather(table, hist_idx, src_off, mov_idx, *, seq, seq_pad,
               hist_chunk, mov_chunk):
    n_hist = hist_idx.shape[0]
    n_mov = mov_idx.shape[0]
    d = table.shape[1]
    info = plsc.get_sparse_core_info()
    nw = info.num_cores * info.num_subcores
    nl = info.num_lanes
    hist_per_w = n_hist // nw
    mov_per_w = n_mov // nw
    n_hchunks = hist_per_w // hist_chunk
    n_mchunks = mov_per_w // mov_chunk
    grp = 4 * seq
    gpad = 4 * seq_pad
    g_per_chunk = hist_chunk // grp
    n_hist_pad = (n_hist // grp) * gpad

    mesh = plsc.VectorSubcoreMesh(core_axis_name="c", subcore_axis_name="s")

    @functools.partial(
        pl.kernel,
        mesh=mesh,
        out_type=(
            jax.ShapeDtypeStruct((n_hist_pad, d), jnp.float32),
            jax.ShapeDtypeStruct((n_mov, d), jnp.float32),
        ),
        scratch_types=[
            pltpu.VMEM((hist_per_w,), jnp.int32),
            pltpu.VMEM((hist_per_w,), jnp.int32),
            pltpu.VMEM((hist_chunk,), jnp.int32),
            pltpu.VMEM((hist_chunk, d), jnp.float32),
            pltpu.VMEM((mov_chunk,), jnp.int32),
            pltpu.VMEM((mov_chunk, d), jnp.float32),
            pltpu.SemaphoreType.DMA,
        ],
        compiler_params=pltpu.CompilerParams(
            use_tc_tiling_on_sc=False, needs_layout_passes=False),
    )
    def k(table_hbm, hidx_hbm, soff_hbm, midx_hbm, hist_out, mov_out,
          nat_v, soff_v, pidx_v, hrows_v, midx_v, mrows_v, sem):
        wid = lax.axis_index("s") * info.num_cores + lax.axis_index("c")
        hbase = wid * hist_per_w
        gbase = wid * (hist_per_w // grp)
        mbase = wid * mov_per_w

        pltpu.sync_copy(hidx_hbm.at[pl.ds(hbase, hist_per_w)], nat_v)
        pltpu.sync_copy(soff_hbm, soff_v)

        def hist_body(i, _):
            local = i * hist_chunk

            def perm_body(v, _):
                p = pl.ds(local + v * nl, nl)
                pidx_v[pl.ds(v * nl, nl)] = plsc.load_gather(
                    nat_v, [soff_v[p]])
                return 0

            lax.fori_loop(0, hist_chunk // nl, perm_body, 0)
            pltpu.async_copy(table_hbm.at[pidx_v], hrows_v, sem).wait()
            for kk in range(g_per_chunk):
                pltpu.sync_copy(
                    hrows_v.at[pl.ds(kk * grp, grp)],
                    hist_out.at[pl.ds(
                        (gbase + i * g_per_chunk + kk) * gpad, grp)])
            return 0

        lax.fori_loop(0, n_hchunks, hist_body, 0)

        def mov_body(i, _):
            base = mbase + i * mov_chunk
            pltpu.sync_copy(midx_hbm.at[pl.ds(base, mov_chunk)], midx_v)
            pltpu.async_copy(table_hbm.at[midx_v], mrows_v, sem).wait()
            pltpu.sync_copy(mrows_v, mov_out.at[pl.ds(base, mov_chunk)])
            return 0

        lax.fori_loop(0, n_mchunks, mov_body, 0)

    return k(table, hist_idx, src_off, mov_idx)


def _tc_body(hist_ref, rat_ref, mov_ref, w1_ref, b1_ref, w2_ref, b2_ref,
             w3_ref, b3_ref, out_ref, *, gsize, seq, seq_pad, dim):
    f32 = jnp.float32
    x = hist_ref[...]
    r = jnp.transpose(rat_ref[...], (0, 2, 1))

    lanes = 4 * dim
    lane_id = lax.broadcasted_iota(jnp.int32, (4, lanes), 1)
    row_id = lax.broadcasted_iota(jnp.int32, (4, lanes), 0)
    sel = jnp.where(lane_id // dim == row_id, f32(1), f32(0))
    r_exp = jnp.dot(r.reshape(gsize * seq, 4), sel,
                    preferred_element_type=f32).reshape(gsize, seq, lanes)
    if seq_pad > seq:
        r_exp = jnp.concatenate(
            [r_exp, jnp.zeros((gsize, seq_pad - seq, lanes), f32)], axis=1)

    inf = f32(jnp.inf)
    l_id = lax.broadcasted_iota(jnp.int32, (gsize, seq_pad, lanes), 1)
    valid = l_id < seq
    wf = x * r_exp
    w = jnp.where(valid, wf, inf)

    mn = jnp.min(w, axis=1)
    mx = jnp.max(jnp.where(valid, wf, -inf), axis=1)
    mean = jnp.sum(jnp.where(valid, wf, f32(0)), axis=1) * f32(1.0 / seq)

    c = jnp.zeros((gsize, seq_pad, lanes), dtype=f32)
    for j in range(seq):
        c = c + jnp.where(w[:, j:j + 1, :] <= w, f32(1), f32(0))
    lo_rank = f32((seq - 1) // 2 + 1)
    hi_rank = f32(seq // 2 + 1)
    s_lo = jnp.min(jnp.where(c >= lo_rank, w, inf), axis=1)
    s_hi = jnp.min(jnp.where(c >= hi_rank, w, inf), axis=1)
    med = f32(0.5) * (s_lo + s_hi)

    def unfold(s):
        parts = [s[:, rr * dim:(rr + 1) * dim].reshape(gsize, 1, dim)
                 for rr in range(4)]
        return jnp.concatenate(parts, axis=1).reshape(4 * gsize, dim)

    u = jnp.concatenate(
        [unfold(mn), unfold(mx), unfold(mean), unfold(med)], axis=1)
    u = u * lax.rsqrt(jnp.sum(u * u, axis=1, keepdims=True))

    xin = jnp.concatenate([u, unfold(mov_ref[...])], axis=1)
    h = jnp.dot(xin, w1_ref[...], preferred_element_type=f32) + b1_ref[...]
    h = jnp.maximum(h, f32(0))
    h = jnp.dot(h, w2_ref[...], preferred_element_type=f32) + b2_ref[...]
    h = jnp.maximum(h, f32(0))
    o = jnp.dot(h, w3_ref[...], preferred_element_type=f32) + b3_ref[...]
    out_ref[...] = f32(1) / (f32(1) + jnp.exp(-o))


def _tc_pool_mlp(hist_fold, rat_fold, mov_fold, W1, b1, W2, b2, W3, b3,
                 *, gsize):
    gtot, seq_pad, lanes = hist_fold.shape
    seq = rat_fold.shape[2]
    dim = lanes // 4
    batch = 4 * gtot
    nblocks = gtot // gsize
    body = functools.partial(_tc_body, gsize=gsize, seq=seq,
                             seq_pad=seq_pad, dim=dim)
    return pl.pallas_call(
        body,
        grid=(nblocks,),
        in_specs=[
            pl.BlockSpec((gsize, seq_pad, lanes), lambda i: (i, 0, 0)),
            pl.BlockSpec((gsize, 4, seq), lambda i: (i, 0, 0)),
            pl.BlockSpec((gsize, lanes), lambda i: (i, 0)),
            pl.BlockSpec(W1.shape, lambda i: (0, 0)),
            pl.BlockSpec((1, W1.shape[1]), lambda i: (0, 0)),
            pl.BlockSpec(W2.shape, lambda i: (0, 0)),
            pl.BlockSpec((1, W2.shape[1]), lambda i: (0, 0)),
            pl.BlockSpec(W3.shape, lambda i: (0, 0)),
            pl.BlockSpec((1, 1), lambda i: (0, 0)),
        ],
        out_specs=pl.BlockSpec((4 * gsize, 1), lambda i: (i, 0)),
        out_shape=jax.ShapeDtypeStruct((batch, 1), jnp.float32),
        compiler_params=pltpu.CompilerParams(
            dimension_semantics=("parallel",)),
    )(hist_fold, rat_fold, mov_fold, W1, b1.reshape(1, -1), W2,
      b2.reshape(1, -1), W3, b3.reshape(1, 1))


def kernel(user_hist_indices, user_hist_ratings, movie_indices, movie_table,
           W1, b1, W2, b2, W3, b3):
    batch, seq = user_hist_indices.shape
    dim = movie_table.shape[1]

    info = plsc.get_sparse_core_info()
    nw = info.num_cores * info.num_subcores
    seq_pad = (seq + 7) // 8 * 8

    nh = 2
    hb = batch // nh
    ghalf = hb // 4

    per_w = (hb * seq) // nw
    grp = 4 * seq
    n = jnp.arange(per_w, dtype=jnp.int32)
    src_off = (n // grp) * grp + (n % 4) * seq + (n % grp) // 4

    outs = []
    for h in range(nh):
        sl = slice(h * hb, (h + 1) * hb)
        hist_rows, mov_emb = _sc_gather(
            movie_table,
            user_hist_indices[sl].astype(jnp.int32).reshape(-1),
            src_off, movie_indices[sl].astype(jnp.int32),
            seq=seq, seq_pad=seq_pad, hist_chunk=1600,
            mov_chunk=hb // nw)
        hist_fold = hist_rows.reshape(ghalf, seq_pad, 4 * dim)
        mov_fold = mov_emb.reshape(ghalf, 4 * dim)
        rat_fold = user_hist_ratings[sl].reshape(ghalf, 4, seq)
        outs.append(_tc_pool_mlp(hist_fold, rat_fold, mov_fold,
                                 W1, b1, W2, b2, W3, b3, gsize=64))
    return jnp.concatenate(outs, axis=0)


# speedup vs baseline: 1.0570x; 1.0039x over previous
import functools

import jax
import jax.numpy as jnp
from jax import lax
from jax.experimental import pallas as pl
from jax.experimental.pallas import tpu as pltpu
from jax.experimental.pallas import tpu_sc as plsc


def _sc_gather(table, hist_idx, src_off, mov_idx, *, seq, seq_pad,
               hist_chunk, mov_chunk):
    n_hist = hist_idx.shape[0]
    n_mov = mov_idx.shape[0]
    d = table.shape[1]
    info = plsc.get_sparse_core_info()
    nw = info.num_cores * info.num_subcores
    nl = info.num_lanes
    hist_per_w = n_hist // nw
    mov_per_w = n_mov // nw
    n_hchunks = hist_per_w // hist_chunk
    n_mchunks = mov_per_w // mov_chunk
    grp = 4 * seq
    gpad = 4 * seq_pad
    g_per_chunk = hist_chunk // grp
    n_hist_pad = (n_hist // grp) * gpad

    mesh = plsc.VectorSubcoreMesh(core_axis_name="c", subcore_axis_name="s")

    @functools.partial(
        pl.kernel,
        mesh=mesh,
        out_type=(
            jax.ShapeDtypeStruct((n_hist_pad, d), jnp.float32),
            jax.ShapeDtypeStruct((n_mov, d), jnp.float32),
        ),
        scratch_types=[
            pltpu.VMEM((hist_per_w,), jnp.int32),
            pltpu.VMEM((hist_per_w,), jnp.int32),
            pltpu.VMEM((hist_chunk,), jnp.int32),
            pltpu.VMEM((hist_chunk, d), jnp.float32),
            pltpu.VMEM((mov_chunk,), jnp.int32),
            pltpu.VMEM((mov_chunk, d), jnp.float32),
            pltpu.SemaphoreType.DMA,
        ],
        compiler_params=pltpu.CompilerParams(
            use_tc_tiling_on_sc=False, needs_layout_passes=False),
    )
    def k(table_hbm, hidx_hbm, soff_hbm, midx_hbm, hist_out, mov_out,
          nat_v, soff_v, pidx_v, hrows_v, midx_v, mrows_v, sem):
        wid = lax.axis_index("s") * info.num_cores + lax.axis_index("c")
        hbase = wid * hist_per_w
        gbase = wid * (hist_per_w // grp)
        mbase = wid * mov_per_w

        pltpu.sync_copy(hidx_hbm.at[pl.ds(hbase, hist_per_w)], nat_v)
        pltpu.sync_copy(soff_hbm, soff_v)

        def hist_body(i, _):
            local = i * hist_chunk

            def perm_body(v, _):
                p = pl.ds(local + v * nl, nl)
                pidx_v[pl.ds(v * nl, nl)] = plsc.load_gather(
                    nat_v, [soff_v[p]])
                return 0

            lax.fori_loop(0, hist_chunk // nl, perm_body, 0)
            pltpu.async_copy(table_hbm.at[pidx_v], hrows_v, sem).wait()
            for kk in range(g_per_chunk):
                pltpu.sync_copy(
                    hrows_v.at[pl.ds(kk * grp, grp)],
                    hist_out.at[pl.ds(
                        (gbase + i * g_per_chunk + kk) * gpad, grp)])
            return 0

        lax.fori_loop(0, n_hchunks, hist_body, 0)

        def mov_body(i, _):
            base = mbase + i * mov_chunk
            pltpu.sync_copy(midx_hbm.at[pl.ds(base, mov_chunk)], midx_v)
            pltpu.async_copy(table_hbm.at[midx_v], mrows_v, sem).wait()
            pltpu.sync_copy(mrows_v, mov_out.at[pl.ds(base, mov_chunk)])
            return 0

        lax.fori_loop(0, n_mchunks, mov_body, 0)

    return k(table, hist_idx, src_off, mov_idx)


def _tc_body(hist_ref, rat_ref, mov_ref, w1_ref, b1_ref, w2_ref, b2_ref,
             w3_ref, b3_ref, out_ref, *, gsize, seq, seq_pad, dim):
    f32 = jnp.float32
    x = hist_ref[...]
    r = jnp.transpose(rat_ref[...], (0, 2, 1))

    lanes = 4 * dim
    lane_id = lax.broadcasted_iota(jnp.int32, (4, lanes), 1)
    row_id = lax.broadcasted_iota(jnp.int32, (4, lanes), 0)
    sel = jnp.where(lane_id // dim == row_id, f32(1), f32(0))
    r_exp = jnp.dot(r.reshape(gsize * seq, 4), sel,
                    preferred_element_type=f32).reshape(gsize, seq, lanes)
    if seq_pad > seq:
        r_exp = jnp.concatenate(
            [r_exp, jnp.zeros((gsize, seq_pad - seq, lanes), f32)], axis=1)

    inf = f32(jnp.inf)
    l_id = lax.broadcasted_iota(jnp.int32, (gsize, seq_pad, lanes), 1)
    valid = l_id < seq
    wf = x * r_exp
    w = jnp.where(valid, wf, inf)

    mn = jnp.min(w, axis=1)
    mx = jnp.max(jnp.where(valid, wf, -inf), axis=1)
    mean = jnp.sum(jnp.where(valid, wf, f32(0)), axis=1) * f32(1.0 / seq)

    c = jnp.zeros((gsize, seq_pad, lanes), dtype=f32)
    for j in range(seq):
        c = c + jnp.where(w[:, j:j + 1, :] <= w, f32(1), f32(0))
    lo_rank = f32((seq - 1) // 2 + 1)
    hi_rank = f32(seq // 2 + 1)
    s_lo = jnp.min(jnp.where(c >= lo_rank, w, inf), axis=1)
    s_hi = jnp.min(jnp.where(c >= hi_rank, w, inf), axis=1)
    med = f32(0.5) * (s_lo + s_hi)

    def unfold(s):
        parts = [s[:, rr * dim:(rr + 1) * dim].reshape(gsize, 1, dim)
                 for rr in range(4)]
        return jnp.concatenate(parts, axis=1).reshape(4 * gsize, dim)

    u = jnp.concatenate(
        [unfold(mn), unfold(mx), unfold(mean), unfold(med)], axis=1)
    u = u * lax.rsqrt(jnp.sum(u * u, axis=1, keepdims=True))

    xin = jnp.concatenate([u, unfold(mov_ref[...])], axis=1)
    h = jnp.dot(xin, w1_ref[...], preferred_element_type=f32) + b1_ref[...]
    h = jnp.maximum(h, f32(0))
    h = jnp.dot(h, w2_ref[...], preferred_element_type=f32) + b2_ref[...]
    h = jnp.maximum(h, f32(0))
    o = jnp.dot(h, w3_ref[...], preferred_element_type=f32) + b3_ref[...]
    out_ref[...] = f32(1) / (f32(1) + jnp.exp(-o))


def _tc_pool_mlp(hist_fold, rat_fold, mov_fold, W1, b1, W2, b2, W3, b3,
                 *, gsize):
    gtot, seq_pad, lanes = hist_fold.shape
    seq = rat_fold.shape[2]
    dim = lanes // 4
    batch = 4 * gtot
    nblocks = gtot // gsize
    body = functools.partial(_tc_body, gsize=gsize, seq=seq,
                             seq_pad=seq_pad, dim=dim)
    return pl.pallas_call(
        body,
        grid=(nblocks,),
        in_specs=[
            pl.BlockSpec((gsize, seq_pad, lanes), lambda i: (i, 0, 0)),
            pl.BlockSpec((gsize, 4, seq), lambda i: (i, 0, 0)),
            pl.BlockSpec((gsize, lanes), lambda i: (i, 0)),
            pl.BlockSpec(W1.shape, lambda i: (0, 0)),
            pl.BlockSpec((1, W1.shape[1]), lambda i: (0, 0)),
            pl.BlockSpec(W2.shape, lambda i: (0, 0)),
            pl.BlockSpec((1, W2.shape[1]), lambda i: (0, 0)),
            pl.BlockSpec(W3.shape, lambda i: (0, 0)),
            pl.BlockSpec((1, 1), lambda i: (0, 0)),
        ],
        out_specs=pl.BlockSpec((4 * gsize, 1), lambda i: (i, 0)),
        out_shape=jax.ShapeDtypeStruct((batch, 1), jnp.float32),
        compiler_params=pltpu.CompilerParams(
            dimension_semantics=("parallel",)),
    )(hist_fold, rat_fold, mov_fold, W1, b1.reshape(1, -1), W2,
      b2.reshape(1, -1), W3, b3.reshape(1, 1))


def kernel(user_hist_indices, user_hist_ratings, movie_indices, movie_table,
           W1, b1, W2, b2, W3, b3):
    batch, seq = user_hist_indices.shape
    dim = movie_table.shape[1]

    info = plsc.get_sparse_core_info()
    nw = info.num_cores * info.num_subcores
    seq_pad = (seq + 7) // 8 * 8

    nh = 4
    hb = batch // nh
    ghalf = hb // 4

    per_w = (hb * seq) // nw
    grp = 4 * seq
    n = jnp.arange(per_w, dtype=jnp.int32)
    src_off = (n // grp) * grp + (n % 4) * seq + (n % grp) // 4

    outs = []
    for h in range(nh):
        sl = slice(h * hb, (h + 1) * hb)
        hist_rows, mov_emb = _sc_gather(
            movie_table,
            user_hist_indices[sl].astype(jnp.int32).reshape(-1),
            src_off, movie_indices[sl].astype(jnp.int32),
            seq=seq, seq_pad=seq_pad, hist_chunk=1600,
            mov_chunk=hb // nw)
        hist_fold = hist_rows.reshape(ghalf, seq_pad, 4 * dim)
        mov_fold = mov_emb.reshape(ghalf, 4 * dim)
        rat_fold = user_hist_ratings[sl].reshape(ghalf, 4, seq)
        outs.append(_tc_pool_mlp(hist_fold, rat_fold, mov_fold,
                                 W1, b1, W2, b2, W3, b3, gsize=64))
    return jnp.concatenate(outs, axis=0)
